# baseline (speedup 1.0000x reference)
import functools

import jax
import jax.numpy as jnp
from jax import lax
from jax.experimental import pallas as pl
from jax.experimental.pallas import tpu as pltpu
from jax.experimental.pallas import tpu_sc as plsc

HID = 4096
E = 64
K = 8
TB = 1024
NSUB = 16
NIDX = 16384 * K
IDX_PER_W = NIDX // NSUB
LANES = 16
EPAD = 128


def _gating_block(x_ref, w_ref, gates_ref, idx_ref, mval_ref, midx_ref):
    x = x_ref[...]
    w = w_ref[...]
    work = jax.lax.dot_general(
        w, x, (((1,), (1,)), ((), ())),
        preferred_element_type=jnp.float32,
        precision=jax.lax.Precision.DEFAULT)

    rows = jax.lax.broadcasted_iota(jnp.int32, (E, TB), 0)
    neg = jnp.float32(-jnp.inf)
    for k in range(K):
        m = jnp.max(work, axis=0, keepdims=True)
        a = jnp.min(jnp.where(work == m, rows, E), axis=0, keepdims=True)
        mval_ref[k:k + 1, :] = m
        midx_ref[k:k + 1, :] = a
        work = jnp.where(rows == a, neg, work)

    rowmax = mval_ref[0:1, :]
    sel_exp = jnp.exp(mval_ref[...] - rowmax)
    rest = jnp.sum(jnp.exp(work - rowmax), axis=0, keepdims=True)
    sel_sum = jnp.sum(sel_exp, axis=0, keepdims=True)
    z = sel_sum + rest
    gates_ref[...] = (sel_exp / z) / (sel_sum / z + 1e-8)
    idx_ref[...] = midx_ref[...]


@functools.partial(
    pl.kernel,
    mesh=plsc.VectorSubcoreMesh(
        core_axis_name="c", subcore_axis_name="s", num_cores=1),
    out_type=jax.ShapeDtypeStruct((LANES,), jnp.float32),
    scratch_types=[
        pltpu.VMEM((IDX_PER_W,), jnp.int32),
        pltpu.VMEM((IDX_PER_W,), jnp.float32),
        pltpu.VMEM((EPAD,), jnp.float32),
        pltpu.VMEM((EPAD,), jnp.float32),
        pltpu.VMEM((LANES,), jnp.float32),
        pltpu.VMEM_SHARED((EPAD,), jnp.float32),
    ],
)
def _sc_loss(idx_hbm, loss_hbm, idx_v, ones_v, zeros_v, chist_v, lvec_v,
             shared_h):
    wid = lax.axis_index("s")
    for j in range(EPAD // LANES):
        zeros_v[pl.ds(j * LANES, LANES)] = jnp.zeros((LANES,), jnp.float32)

    @pl.when(wid == 0)
    def _init_shared():
        pltpu.sync_copy(zeros_v, shared_h)
    plsc.subcore_barrier()

    pltpu.sync_copy(idx_hbm.at[pl.ds(wid * IDX_PER_W, IDX_PER_W)], idx_v)

    def body(i, carry):
        ones_v[pl.ds(i * LANES, LANES)] = jnp.full((LANES,), 1.0, jnp.float32)
        return carry

    lax.fori_loop(0, IDX_PER_W // LANES, body, 0)

    pltpu.sync_copy(ones_v, shared_h.at[idx_v], add=True)
    plsc.subcore_barrier()

    @pl.when(wid == 0)
    def _loss():
        pltpu.sync_copy(shared_h, chist_v)

        def lane_allsum(v):
            dnums = lax.GatherDimensionNumbers(
                offset_dims=(), collapsed_slice_dims=(0,),
                start_index_map=(0,))
            base = lax.iota(jnp.int32, LANES)
            for shift in (8, 4, 2, 1):
                perm = jnp.bitwise_and(base + shift, LANES - 1)
                v = v + lax.gather(
                    v, perm[:, None], dimension_numbers=dnums,
                    slice_sizes=(1,),
                    mode=lax.GatherScatterMode.PROMISE_IN_BOUNDS)
            return v

        c = [chist_v[pl.ds(j * LANES, LANES)] for j in range(E // LANES)]
        total = lane_allsum(c[0] + c[1] + c[2] + c[3])
        u = [cj / total for cj in c]
        mean_u = lane_allsum(u[0] + u[1] + u[2] + u[3]) * (1.0 / E)
        d = [(uj - mean_u) * (uj - mean_u) for uj in u]
        var_u = lane_allsum(d[0] + d[1] + d[2] + d[3]) * (1.0 / (E - 1))
        cv = var_u / (mean_u + 1e-8)
        lvec_v[...] = cv * cv
        pltpu.sync_copy(lvec_v, loss_hbm)


def kernel(x, W):
    B_, S_, H_ = x.shape
    T = B_ * S_
    xf = x.reshape(T, H_)
    gates_kt, idx_kt = pl.pallas_call(
        _gating_block,
        grid=(T // TB,),
        in_specs=[
            pl.BlockSpec((TB, H_), lambda i: (i, 0)),
            pl.BlockSpec((E, H_), lambda i: (0, 0)),
        ],
        out_specs=[
            pl.BlockSpec((K, TB), lambda i: (0, i)),
            pl.BlockSpec((K, TB), lambda i: (0, i)),
        ],
        out_shape=[
            jax.ShapeDtypeStruct((K, T), jnp.float32),
            jax.ShapeDtypeStruct((K, T), jnp.int32),
        ],
        scratch_shapes=[
            pltpu.VMEM((K, TB), jnp.float32),
            pltpu.VMEM((K, TB), jnp.int32),
        ],
        compiler_params=pltpu.CompilerParams(
            dimension_semantics=("arbitrary",)),
    )(xf, W)
    loss_vec = _sc_loss(idx_kt.reshape(NIDX))
    gates = jnp.transpose(gates_kt).reshape(B_, S_, K)
    idx = jnp.transpose(idx_kt).reshape(B_, S_, K)
    return (gates, idx, loss_vec[0])


# speedup vs baseline: 1.0686x; 1.0686x over previous
import functools

import jax
import jax.numpy as jnp
from jax import lax
from jax.experimental import pallas as pl
from jax.experimental.pallas import tpu as pltpu
from jax.experimental.pallas import tpu_sc as plsc

HID = 4096
E = 64
K = 8
TB = 1024
NSTEPS = 16384 // TB
LANES = 16


def _gating_block(x_ref, w_ref, gates_ref, idx_ref, cpart_ref,
                  mval_ref, midx_ref):
    x = x_ref[...]
    w = w_ref[...]
    work = jax.lax.dot_general(
        w, x, (((1,), (1,)), ((), ())),
        preferred_element_type=jnp.float32,
        precision=jax.lax.Precision.DEFAULT)

    rows = jax.lax.broadcasted_iota(jnp.int32, (E, TB), 0)
    neg = jnp.float32(-jnp.inf)
    for k in range(K):
        m = jnp.max(work, axis=0, keepdims=True)
        a = jnp.min(jnp.where(work == m, rows, E), axis=0, keepdims=True)
        mval_ref[k:k + 1, :] = m
        midx_ref[k:k + 1, :] = a
        work = jnp.where(rows == a, neg, work)

    rowmax = mval_ref[0:1, :]
    sel_exp = jnp.exp(mval_ref[...] - rowmax)
    rest = jnp.sum(jnp.exp(work - rowmax), axis=0, keepdims=True)
    sel_sum = jnp.sum(sel_exp, axis=0, keepdims=True)
    z = sel_sum + rest
    gates_ref[...] = (sel_exp / z) / (sel_sum / z + 1e-8)
    idx_ref[...] = midx_ref[...]

    sel = (work == neg).astype(jnp.float32)
    cpart_ref[...] = jnp.sum(sel, axis=1, keepdims=True).reshape(1, 1, E)


@functools.partial(
    pl.kernel,
    mesh=plsc.VectorSubcoreMesh(
        core_axis_name="c", subcore_axis_name="s", num_cores=1),
    out_type=jax.ShapeDtypeStruct((LANES,), jnp.float32),
    scratch_types=[
        pltpu.VMEM((NSTEPS * E,), jnp.float32),
        pltpu.VMEM((LANES,), jnp.float32),
    ],
)
def _sc_loss(cpart_hbm, loss_hbm, cpart_v, lvec_v):
    wid = lax.axis_index("s")

    @pl.when(wid == 0)
    def _loss():
        pltpu.sync_copy(cpart_hbm, cpart_v)
        nvec = E // LANES
        acc = [jnp.zeros((LANES,), jnp.float32) for _ in range(nvec)]
        for s in range(NSTEPS):
            for j in range(nvec):
                acc[j] = acc[j] + cpart_v[pl.ds(s * E + j * LANES, LANES)]

        def lane_allsum(v):
            dnums = lax.GatherDimensionNumbers(
                offset_dims=(), collapsed_slice_dims=(0,),
                start_index_map=(0,))
            base = lax.iota(jnp.int32, LANES)
            for shift in (8, 4, 2, 1):
                perm = jnp.bitwise_and(base + shift, LANES - 1)
                v = v + lax.gather(
                    v, perm[:, None], dimension_numbers=dnums,
                    slice_sizes=(1,),
                    mode=lax.GatherScatterMode.PROMISE_IN_BOUNDS)
            return v

        total = lane_allsum(acc[0] + acc[1] + acc[2] + acc[3])
        u = [cj / total for cj in acc]
        mean_u = lane_allsum(u[0] + u[1] + u[2] + u[3]) * (1.0 / E)
        d = [(uj - mean_u) * (uj - mean_u) for uj in u]
        var_u = lane_allsum(d[0] + d[1] + d[2] + d[3]) * (1.0 / (E - 1))
        cv = var_u / (mean_u + 1e-8)
        lvec_v[...] = cv * cv
        pltpu.sync_copy(lvec_v, loss_hbm)


def kernel(x, W):
    B_, S_, H_ = x.shape
    T = B_ * S_
    xf = x.reshape(T, H_)
    gates_kt, idx_kt, cparts = pl.pallas_call(
        _gating_block,
        grid=(T // TB,),
        in_specs=[
            pl.BlockSpec((TB, H_), lambda i: (i, 0)),
            pl.BlockSpec((E, H_), lambda i: (0, 0)),
        ],
        out_specs=[
            pl.BlockSpec((K, TB), lambda i: (0, i)),
            pl.BlockSpec((K, TB), lambda i: (0, i)),
            pl.BlockSpec((1, 1, E), lambda i: (i, 0, 0)),
        ],
        out_shape=[
            jax.ShapeDtypeStruct((K, T), jnp.float32),
            jax.ShapeDtypeStruct((K, T), jnp.int32),
            jax.ShapeDtypeStruct((NSTEPS, 1, E), jnp.float32),
        ],
        scratch_shapes=[
            pltpu.VMEM((K, TB), jnp.float32),
            pltpu.VMEM((K, TB), jnp.int32),
        ],
        compiler_params=pltpu.CompilerParams(
            dimension_semantics=("arbitrary",)),
    )(xf, W)
    loss_vec = _sc_loss(cparts.reshape(NSTEPS * E))
    gates = jnp.transpose(gates_kt).reshape(B_, S_, K)
    idx = jnp.transpose(idx_kt).reshape(B_, S_, K)
    return (gates, idx, loss_vec[0])


# speedup vs baseline: 1.3017x; 1.2182x over previous
import jax
import jax.numpy as jnp
from jax.experimental import pallas as pl
from jax.experimental.pallas import tpu as pltpu

HID = 4096
E = 64
K = 8
TB = 1024


def _gating_block(x_ref, w_ref, gates_ref, idx_ref, loss_ref,
                  counts_ref, mval_ref, midx_ref):
    i = pl.program_id(0)
    nsteps = pl.num_programs(0)

    @pl.when(i == 0)
    def _init():
        counts_ref[...] = jnp.zeros_like(counts_ref)

    x = x_ref[...]
    w = w_ref[...]
    work = jax.lax.dot_general(
        w, x, (((1,), (1,)), ((), ())),
        preferred_element_type=jnp.float32,
        precision=jax.lax.Precision.DEFAULT)

    rows = jax.lax.broadcasted_iota(jnp.int32, (E, TB), 0)
    neg = jnp.float32(-jnp.inf)
    for k in range(K):
        m = jnp.max(work, axis=0, keepdims=True)
        a = jnp.min(jnp.where(work == m, rows, E), axis=0, keepdims=True)
        mval_ref[k:k + 1, :] = m
        midx_ref[k:k + 1, :] = a
        work = jnp.where(rows == a, neg, work)

    rowmax = mval_ref[0:1, :]
    sel_exp = jnp.exp(mval_ref[...] - rowmax)
    rest = jnp.sum(jnp.exp(work - rowmax), axis=0, keepdims=True)
    sel_sum = jnp.sum(sel_exp, axis=0, keepdims=True)
    z = sel_sum + rest
    gates_ref[...] = (sel_exp / z) / (sel_sum / z + 1e-8)
    idx_ref[...] = midx_ref[...]

    sel = (work == neg).astype(jnp.float32)
    counts_ref[...] += jnp.sum(sel, axis=1, keepdims=True)

    @pl.when(i == nsteps - 1)
    def _loss():
        counts = counts_ref[...]
        total = jnp.sum(counts, axis=0, keepdims=True)
        usage = counts / total
        mean_u = jnp.sum(usage, axis=0, keepdims=True) / E
        var_u = jnp.sum((usage - mean_u) ** 2, axis=0, keepdims=True) / (E - 1)
        loss_ref[...] = (var_u / (mean_u + 1e-8)) ** 2


def kernel(x, W):
    B_, S_, H_ = x.shape
    T = B_ * S_
    xf = x.reshape(T, H_)
    gates_kt, idx_kt, loss = pl.pallas_call(
        _gating_block,
        grid=(T // TB,),
        in_specs=[
            pl.BlockSpec((TB, H_), lambda i: (i, 0)),
            pl.BlockSpec((E, H_), lambda i: (0, 0)),
        ],
        out_specs=[
            pl.BlockSpec((K, TB), lambda i: (0, i)),
            pl.BlockSpec((K, TB), lambda i: (0, i)),
            pl.BlockSpec((1, 1), lambda i: (0, 0)),
        ],
        out_shape=[
            jax.ShapeDtypeStruct((K, T), jnp.float32),
            jax.ShapeDtypeStruct((K, T), jnp.int32),
            jax.ShapeDtypeStruct((1, 1), jnp.float32),
        ],
        scratch_shapes=[
            pltpu.VMEM((E, 1), jnp.float32),
            pltpu.VMEM((K, TB), jnp.float32),
            pltpu.VMEM((K, TB), jnp.int32),
        ],
        compiler_params=pltpu.CompilerParams(
            dimension_semantics=("arbitrary",)),
    )(xf, W)
    gates = jnp.transpose(gates_kt).reshape(B_, S_, K)
    idx = jnp.transpose(idx_kt).reshape(B_, S_, K)
    return (gates, idx, loss[0, 0])
